# baseline (speedup 1.0000x reference)
import functools

import jax
import jax.numpy as jnp
from jax import lax
from jax.experimental import pallas as pl
from jax.experimental.pallas import tpu as pltpu
from jax.experimental.pallas import tpu_sc as plsc

B, S = 8, 512
F32 = jnp.float32

TC_B = 5
NSC = B - TC_B
NW = 32
SLABS = NSC * S
SLABS_PW = SLABS // NW
GULP = 16
NGULP = SLABS_PW // GULP


_GATHER_DNUMS = lax.GatherDimensionNumbers(
    offset_dims=(), collapsed_slice_dims=(0,), start_index_map=(0,))


def _lane_allmax(v, lane):
    for d in (8, 4, 2, 1):
        idx = lane ^ d
        p = lax.gather(v, idx[:, None], _GATHER_DNUMS, slice_sizes=(1,),
                       mode=lax.GatherScatterMode.PROMISE_IN_BOUNDS)
        v = jnp.maximum(v, p)
    return v


def _sc_caus_body(rel_hbm, out_hbm, buf, res):
    wid = lax.axis_index("s") * 2 + lax.axis_index("c")
    slab0 = TC_B * S + wid * SLABS_PW
    lane = lax.broadcasted_iota(jnp.int32, (16,), 0)

    def gulp_body(g, carry):
        first = slab0 + g * GULP
        pltpu.sync_copy(rel_hbm.at[pl.ds(first * 16, GULP * 16)], buf)
        rvec = jnp.zeros((16,), F32)
        for s in range(GULP):
            i = (first + s) & (S - 1)
            acc = jnp.zeros((16,), F32)
            for jblk in range(4):
                d = i - jblk * 128
                nfull = jnp.clip(d >> 4, 0, 8)

                def lv_body(lv, a, _s=s, _jb=jblk):
                    off = lv * 16
                    for c in range(4):
                        v = buf[_s * 16 + _jb * 4 + c, pl.ds(off, 16)]
                        a = jnp.maximum(a, v)
                    return a

                acc = lax.fori_loop(0, nfull, lv_body, acc)
                rem = jnp.where(nfull < 8, d - (nfull << 4), 0)
                poff = jnp.minimum(nfull, 7) * 16
                pmask = lane < rem
                for c in range(4):
                    v = buf[s * 16 + jblk * 4 + c, pl.ds(poff, 16)]
                    acc = jnp.maximum(acc, jnp.where(pmask, v, 0.0))
            amax = _lane_allmax(acc, lane)
            rvec = jnp.where((lane == s) & (amax > 0.5), 1.0, rvec)
        res[pl.ds(g * GULP, GULP)] = rvec
        return carry

    lax.fori_loop(0, NGULP, gulp_body, 0)
    pltpu.sync_copy(res, out_hbm.at[pl.ds(wid * SLABS_PW, SLABS_PW)])


def _sc_caus(rel_rows):
    mesh = plsc.VectorSubcoreMesh(core_axis_name="c", subcore_axis_name="s")
    k = functools.partial(
        pl.kernel,
        mesh=mesh,
        out_type=jax.ShapeDtypeStruct((SLABS,), F32),
        scratch_types=[
            pltpu.VMEM((GULP * 16, 128), F32),
            pltpu.VMEM((SLABS_PW,), F32),
        ],
    )(_sc_caus_body)
    return k(rel_rows)


def _shift_fwd(x, d, fill, lane):
    r = pltpu.roll(x, d, 1)
    return jnp.where(lane >= d, r, fill)


def _shift_bwd(x, d, fill, lane):
    r = pltpu.roll(x, S - d, 1)
    return jnp.where(lane < S - d, r, fill)


def _channels(view):
    return [jnp.concatenate([view[:, 4 * k + c, :] for k in range(4)], axis=1)
            for c in range(4)]


def _body(rel_ref, av_ref, ov_ref, ea_ref, eo_ref, scc_ref,
          w1t_ref, b1c_ref, w2t_ref, b2c_ref, w3t_ref, b3c_ref,
          cav_ref, cov_ref, caus_sc):
    b = pl.program_id(0)

    x = rel_ref[0]
    i_iota = jax.lax.broadcasted_iota(jnp.int32, (S, 16, 128), 0)
    m_iota = jax.lax.broadcasted_iota(jnp.int32, (S, 16, 128), 1)
    l_iota = jax.lax.broadcasted_iota(jnp.int32, (S, 16, 128), 2)
    j = ((m_iota >> 2) << 7) + l_iota
    hit = (x > 0.5) & (j < i_iota)
    caus_sc[b] = jnp.any(hit, axis=(1, 2)).astype(F32)[:, None]

    @pl.when(b == TC_B - 1)
    def _main():
        a0, a1, a2, a3 = _channels(av_ref[...])
        o0, o1, o2, o3 = _channels(ov_ref[...])

        feats = [a0, a1, a2, a3, o0, o1, o2, o3]
        xall = jnp.concatenate(
            [jnp.reshape(f, (1, B * S)) for f in feats], axis=0)
        h = jnp.dot(w1t_ref[...], xall, preferred_element_type=F32) + b1c_ref[...]
        h = jnp.maximum(h, 0.0)
        h = jnp.dot(w2t_ref[...], h, preferred_element_type=F32) + b2c_ref[...]
        h = jnp.maximum(h, 0.0)
        z = jnp.dot(w3t_ref[...], h, preferred_element_type=F32) + b3c_ref[...]
        score = jax.nn.sigmoid(jnp.reshape(z, (B, S)))
        f1 = jnp.where(score < 0.5, 2.0 * score, 1.0)

        def imp(c0, c1, c2, c3):
            m = jnp.maximum(jnp.maximum(c0, c1), jnp.maximum(c2, c3))
            e0, e1 = jnp.exp(c0 - m), jnp.exp(c1 - m)
            e2, e3 = jnp.exp(c2 - m), jnp.exp(c3 - m)
            den = ((e0 + e1) + e2) + e3
            return (e0 / den + e1 / den) > 0.5

        imp_asp = imp(a0, a1, a2, a3)
        imp_op = imp(o0, o1, o2, o3)

        ri = jax.lax.broadcasted_iota(jnp.int32, (S, S), 0)
        ci = jax.lax.broadcasted_iota(jnp.int32, (S, S), 1)
        band = (jnp.abs(ri - ci) <= 3).astype(F32)
        ea = (ea_ref[...] > 0).astype(F32)
        eo = (eo_ref[...] > 0).astype(F32)
        near_op = jnp.dot(eo, band, preferred_element_type=F32) > 0.0
        near_as = jnp.dot(ea, band, preferred_element_type=F32) > 0.0

        r2 = imp_asp & (~near_op)
        r3 = imp_op & (~near_as)
        caus = jnp.concatenate(
            [jnp.reshape(caus_sc[...], (TC_B, S)), scc_ref[...]], axis=0) > 0.5

        w2a = jnp.where(r2, 0.3, 1.0)
        w2o = jnp.where(r3, 0.3, 1.0)
        w7 = jnp.where(caus, 0.7, 1.0)
        u_a0, u_a1 = (a0 * f1) * w2a, (a1 * f1) * w2a
        u_o0, u_o1 = (o0 * f1) * w2o, (o1 * f1) * w2o
        t_a2, t_a3 = a2 * f1, a3 * f1
        t_o2, t_o3 = o2 * f1, o3 * f1

        P = jnp.maximum(jnp.maximum(t_a2, t_a3), jnp.maximum(t_o2, t_o3)) > 0.5
        Q1 = jnp.maximum(jnp.maximum(u_a0 * w7, u_a1 * w7),
                         jnp.maximum(u_o0, u_o1)) > 0.5
        Q0 = jnp.maximum(jnp.maximum((u_a0 * 0.1) * w7, (u_a1 * 0.1) * w7),
                         jnp.maximum(u_o0 * 0.1, u_o1 * 0.1)) > 0.5
        F = jnp.maximum(
            jnp.maximum(jnp.maximum(a0, a1), jnp.maximum(a2, a3)),
            jnp.maximum(jnp.maximum(o0, o1), jnp.maximum(o2, o3))) > 0.5

        Rf = (P | Q0).astype(F32)
        Q1f = Q1.astype(F32)
        Ff = F.astype(F32)

        lane = jax.lax.broadcasted_iota(jnp.int32, (B, S), 1)
        base = jnp.maximum(
            jnp.maximum(_shift_bwd(Ff, 1, 0.0, lane),
                        _shift_bwd(Ff, 2, 0.0, lane)),
            jnp.maximum(_shift_fwd(Rf, 1, 0.0, lane),
                        _shift_fwd(Rf, 2, 0.0, lane)))
        q1s1 = _shift_fwd(Q1f, 1, 0.0, lane)
        q1s2 = _shift_fwd(Q1f, 2, 0.0, lane)

        ones = jnp.ones((B, S), F32)
        zeros = jnp.zeros((B, S), F32)
        a11, a12, a21, a22 = q1s1, q1s2, ones, zeros
        c1, c2 = base, zeros
        d = 1
        while d < S:
            b11 = _shift_fwd(a11, d, 1.0, lane)
            b12 = _shift_fwd(a12, d, 0.0, lane)
            b21 = _shift_fwd(a21, d, 0.0, lane)
            b22 = _shift_fwd(a22, d, 1.0, lane)
            bc1 = _shift_fwd(c1, d, 0.0, lane)
            bc2 = _shift_fwd(c2, d, 0.0, lane)
            n11 = jnp.maximum(a11 * b11, a12 * b21)
            n12 = jnp.maximum(a11 * b12, a12 * b22)
            n21 = jnp.maximum(a21 * b11, a22 * b21)
            n22 = jnp.maximum(a21 * b12, a22 * b22)
            nc1 = jnp.maximum(jnp.maximum(a11 * bc1, a12 * bc2), c1)
            nc2 = jnp.maximum(jnp.maximum(a21 * bc1, a22 * bc2), c2)
            a11, a12, a21, a22, c1, c2 = n11, n12, n21, n22, nc1, nc2
            d *= 2

        iso = jnp.where(c1 > 0.5, 1.0, 0.1)

        outs_a = ((u_a0 * iso) * w7, (u_a1 * iso) * w7, t_a2, t_a3)
        outs_o = (u_o0 * iso, u_o1 * iso, t_o2, t_o3)
        for k in range(4):
            sl = slice(128 * k, 128 * (k + 1))
            for c in range(4):
                cav_ref[:, 4 * k + c, :] = outs_a[c][:, sl]
                cov_ref[:, 4 * k + c, :] = outs_o[c][:, sl]


def _to_view(x):
    return (x.reshape(B, 4, 128, 4)
            .transpose(0, 1, 3, 2)
            .reshape(B, 16, 128))


def _from_view(v):
    return (v.reshape(B, 4, 4, 128)
            .transpose(0, 1, 3, 2)
            .reshape(B, S, 4))


def kernel(aspect_logits, opinion_logits, aspect_opinion_relations,
           explicit_aspects, explicit_opinions, W1, b1, W2, b2, W3, b3):
    rel_v = (aspect_opinion_relations.reshape(B, S, 4, 128, 4)
             .transpose(0, 1, 2, 4, 3)
             .reshape(B, S, 16, 128))
    rel_rows = rel_v.reshape(B * S * 16, 128)
    sc_caus = _sc_caus(rel_rows).reshape(NSC, S)

    av = _to_view(aspect_logits)
    ov = _to_view(opinion_logits)
    ea = explicit_aspects.astype(jnp.int32)
    eo = explicit_opinions.astype(jnp.int32)
    w1t = W1.T
    w2t = W2.T
    w3t = W3.T
    b1c = b1.reshape(32, 1)
    b2c = b2.reshape(16, 1)
    b3c = b3.reshape(1, 1)

    full = lambda shape: pl.BlockSpec(shape, lambda b: (0,) * len(shape))
    cav, cov = pl.pallas_call(
        _body,
        grid=(TC_B,),
        in_specs=[
            pl.BlockSpec((1, S, 16, 128), lambda b: (b, 0, 0, 0)),
            full((B, 16, 128)), full((B, 16, 128)),
            full((B, S)), full((B, S)),
            full((NSC, S)),
            full((32, 8)), full((32, 1)),
            full((16, 32)), full((16, 1)),
            full((1, 16)), full((1, 1)),
        ],
        out_specs=(full((B, 16, 128)), full((B, 16, 128))),
        out_shape=(jax.ShapeDtypeStruct((B, 16, 128), F32),
                   jax.ShapeDtypeStruct((B, 16, 128), F32)),
        scratch_shapes=[pltpu.VMEM((TC_B, S, 1), F32)],
    )(rel_v, av, ov, ea, eo, sc_caus, w1t, b1c, w2t, b2c, w3t, b3c)

    return _from_view(cav), _from_view(cov)


# speedup vs baseline: 1.0183x; 1.0183x over previous
import functools

import jax
import jax.numpy as jnp
from jax import lax
from jax.experimental import pallas as pl
from jax.experimental.pallas import tpu as pltpu
from jax.experimental.pallas import tpu_sc as plsc

B, S = 8, 512
F32 = jnp.float32

TC_B = 5
NSC = B - TC_B
NW = 32
SLABS = NSC * S
SLABS_PW = SLABS // NW
GULP = 16
NGULP = SLABS_PW // GULP


_GATHER_DNUMS = lax.GatherDimensionNumbers(
    offset_dims=(), collapsed_slice_dims=(0,), start_index_map=(0,))


def _lane_allmax(v, lane):
    for d in (8, 4, 2, 1):
        idx = lane ^ d
        p = lax.gather(v, idx[:, None], _GATHER_DNUMS, slice_sizes=(1,),
                       mode=lax.GatherScatterMode.PROMISE_IN_BOUNDS)
        v = jnp.maximum(v, p)
    return v


def _sc_caus_body(rel_hbm, out_hbm, buf, res):
    wid = lax.axis_index("s") * 2 + lax.axis_index("c")
    slab0 = TC_B * S + wid * SLABS_PW
    lane = lax.broadcasted_iota(jnp.int32, (16,), 0)
    zero = jnp.zeros((16,), F32)

    def gulp_body(g, carry):
        first = slab0 + g * GULP
        pltpu.sync_copy(rel_hbm.at[pl.ds(first * 16, GULP * 16)], buf)
        rvec = jnp.zeros((16,), F32)
        for s in range(GULP):
            i = (first + s) & (S - 1)
            accs = [zero, zero, zero, zero]
            for jblk in range(4):
                for lv in range(8):
                    mask = lane < (i - (jblk * 128 + lv * 16))
                    for c in range(4):
                        v = buf[s * 16 + jblk * 4 + c, pl.ds(lv * 16, 16)]
                        accs[c] = jnp.maximum(accs[c], jnp.where(mask, v, 0.0))
            acc = jnp.maximum(jnp.maximum(accs[0], accs[1]),
                              jnp.maximum(accs[2], accs[3]))
            amax = _lane_allmax(acc, lane)
            rvec = jnp.where((lane == s) & (amax > 0.5), 1.0, rvec)
        res[pl.ds(g * GULP, GULP)] = rvec
        return carry

    lax.fori_loop(0, NGULP, gulp_body, 0)
    pltpu.sync_copy(res, out_hbm.at[pl.ds(wid * SLABS_PW, SLABS_PW)])


def _sc_caus(rel_rows):
    mesh = plsc.VectorSubcoreMesh(core_axis_name="c", subcore_axis_name="s")
    k = functools.partial(
        pl.kernel,
        mesh=mesh,
        out_type=jax.ShapeDtypeStruct((SLABS,), F32),
        scratch_types=[
            pltpu.VMEM((GULP * 16, 128), F32),
            pltpu.VMEM((SLABS_PW,), F32),
        ],
    )(_sc_caus_body)
    return k(rel_rows)


def _shift_fwd(x, d, fill, lane):
    r = pltpu.roll(x, d, 1)
    return jnp.where(lane >= d, r, fill)


def _shift_bwd(x, d, fill, lane):
    r = pltpu.roll(x, S - d, 1)
    return jnp.where(lane < S - d, r, fill)


def _channels(view):
    return [jnp.concatenate([view[:, 4 * k + c, :] for k in range(4)], axis=1)
            for c in range(4)]


def _tc_caus_body(rel_ref, out_ref):
    x = rel_ref[0]
    i_iota = jax.lax.broadcasted_iota(jnp.int32, (S, 16, 128), 0)
    m_iota = jax.lax.broadcasted_iota(jnp.int32, (S, 16, 128), 1)
    l_iota = jax.lax.broadcasted_iota(jnp.int32, (S, 16, 128), 2)
    j = ((m_iota >> 2) << 7) + l_iota
    hit = (x > 0.5) & (j < i_iota)
    out_ref[0] = jnp.any(hit, axis=(1, 2)).astype(F32)[:, None]


def _main_body(av_ref, ov_ref, ea_ref, eo_ref, tcc_ref, scc_ref,
               w1t_ref, b1c_ref, w2t_ref, b2c_ref, w3t_ref, b3c_ref,
               cav_ref, cov_ref):
    if True:
        a0, a1, a2, a3 = _channels(av_ref[...])
        o0, o1, o2, o3 = _channels(ov_ref[...])

        feats = [a0, a1, a2, a3, o0, o1, o2, o3]
        xall = jnp.concatenate(
            [jnp.reshape(f, (1, B * S)) for f in feats], axis=0)
        h = jnp.dot(w1t_ref[...], xall, preferred_element_type=F32) + b1c_ref[...]
        h = jnp.maximum(h, 0.0)
        h = jnp.dot(w2t_ref[...], h, preferred_element_type=F32) + b2c_ref[...]
        h = jnp.maximum(h, 0.0)
        z = jnp.dot(w3t_ref[...], h, preferred_element_type=F32) + b3c_ref[...]
        score = jax.nn.sigmoid(jnp.reshape(z, (B, S)))
        f1 = jnp.where(score < 0.5, 2.0 * score, 1.0)

        def imp(c0, c1, c2, c3):
            m = jnp.maximum(jnp.maximum(c0, c1), jnp.maximum(c2, c3))
            e0, e1 = jnp.exp(c0 - m), jnp.exp(c1 - m)
            e2, e3 = jnp.exp(c2 - m), jnp.exp(c3 - m)
            den = ((e0 + e1) + e2) + e3
            return (e0 / den + e1 / den) > 0.5

        imp_asp = imp(a0, a1, a2, a3)
        imp_op = imp(o0, o1, o2, o3)

        ri = jax.lax.broadcasted_iota(jnp.int32, (S, S), 0)
        ci = jax.lax.broadcasted_iota(jnp.int32, (S, S), 1)
        band = (jnp.abs(ri - ci) <= 3).astype(F32)
        ea = (ea_ref[...] > 0).astype(F32)
        eo = (eo_ref[...] > 0).astype(F32)
        near_op = jnp.dot(eo, band, preferred_element_type=F32) > 0.0
        near_as = jnp.dot(ea, band, preferred_element_type=F32) > 0.0

        r2 = imp_asp & (~near_op)
        r3 = imp_op & (~near_as)
        caus = jnp.concatenate(
            [jnp.reshape(tcc_ref[...], (TC_B, S)), scc_ref[...]], axis=0) > 0.5

        w2a = jnp.where(r2, 0.3, 1.0)
        w2o = jnp.where(r3, 0.3, 1.0)
        w7 = jnp.where(caus, 0.7, 1.0)
        u_a0, u_a1 = (a0 * f1) * w2a, (a1 * f1) * w2a
        u_o0, u_o1 = (o0 * f1) * w2o, (o1 * f1) * w2o
        t_a2, t_a3 = a2 * f1, a3 * f1
        t_o2, t_o3 = o2 * f1, o3 * f1

        P = jnp.maximum(jnp.maximum(t_a2, t_a3), jnp.maximum(t_o2, t_o3)) > 0.5
        Q1 = jnp.maximum(jnp.maximum(u_a0 * w7, u_a1 * w7),
                         jnp.maximum(u_o0, u_o1)) > 0.5
        Q0 = jnp.maximum(jnp.maximum((u_a0 * 0.1) * w7, (u_a1 * 0.1) * w7),
                         jnp.maximum(u_o0 * 0.1, u_o1 * 0.1)) > 0.5
        F = jnp.maximum(
            jnp.maximum(jnp.maximum(a0, a1), jnp.maximum(a2, a3)),
            jnp.maximum(jnp.maximum(o0, o1), jnp.maximum(o2, o3))) > 0.5

        Rf = (P | Q0).astype(F32)
        Q1f = Q1.astype(F32)
        Ff = F.astype(F32)

        lane = jax.lax.broadcasted_iota(jnp.int32, (B, S), 1)
        base = jnp.maximum(
            jnp.maximum(_shift_bwd(Ff, 1, 0.0, lane),
                        _shift_bwd(Ff, 2, 0.0, lane)),
            jnp.maximum(_shift_fwd(Rf, 1, 0.0, lane),
                        _shift_fwd(Rf, 2, 0.0, lane)))
        q1s1 = _shift_fwd(Q1f, 1, 0.0, lane)
        q1s2 = _shift_fwd(Q1f, 2, 0.0, lane)

        ones = jnp.ones((B, S), F32)
        zeros = jnp.zeros((B, S), F32)
        a11, a12, a21, a22 = q1s1, q1s2, ones, zeros
        c1, c2 = base, zeros
        d = 1
        while d < S:
            b11 = _shift_fwd(a11, d, 1.0, lane)
            b12 = _shift_fwd(a12, d, 0.0, lane)
            b21 = _shift_fwd(a21, d, 0.0, lane)
            b22 = _shift_fwd(a22, d, 1.0, lane)
            bc1 = _shift_fwd(c1, d, 0.0, lane)
            bc2 = _shift_fwd(c2, d, 0.0, lane)
            n11 = jnp.maximum(a11 * b11, a12 * b21)
            n12 = jnp.maximum(a11 * b12, a12 * b22)
            n21 = jnp.maximum(a21 * b11, a22 * b21)
            n22 = jnp.maximum(a21 * b12, a22 * b22)
            nc1 = jnp.maximum(jnp.maximum(a11 * bc1, a12 * bc2), c1)
            nc2 = jnp.maximum(jnp.maximum(a21 * bc1, a22 * bc2), c2)
            a11, a12, a21, a22, c1, c2 = n11, n12, n21, n22, nc1, nc2
            d *= 2

        iso = jnp.where(c1 > 0.5, 1.0, 0.1)

        outs_a = ((u_a0 * iso) * w7, (u_a1 * iso) * w7, t_a2, t_a3)
        outs_o = (u_o0 * iso, u_o1 * iso, t_o2, t_o3)
        for k in range(4):
            sl = slice(128 * k, 128 * (k + 1))
            for c in range(4):
                cav_ref[:, 4 * k + c, :] = outs_a[c][:, sl]
                cov_ref[:, 4 * k + c, :] = outs_o[c][:, sl]


def _to_view(x):
    return (x.reshape(B, 4, 128, 4)
            .transpose(0, 1, 3, 2)
            .reshape(B, 16, 128))


def _from_view(v):
    return (v.reshape(B, 4, 4, 128)
            .transpose(0, 1, 3, 2)
            .reshape(B, S, 4))


def kernel(aspect_logits, opinion_logits, aspect_opinion_relations,
           explicit_aspects, explicit_opinions, W1, b1, W2, b2, W3, b3):
    rel_v = (aspect_opinion_relations.reshape(B, S, 4, 128, 4)
             .transpose(0, 1, 2, 4, 3)
             .reshape(B, S, 16, 128))
    rel_rows = rel_v.reshape(B * S * 16, 128)
    sc_caus = _sc_caus(rel_rows).reshape(NSC, S)

    av = _to_view(aspect_logits)
    ov = _to_view(opinion_logits)
    ea = explicit_aspects.astype(jnp.int32)
    eo = explicit_opinions.astype(jnp.int32)
    w1t = W1.T
    w2t = W2.T
    w3t = W3.T
    b1c = b1.reshape(32, 1)
    b2c = b2.reshape(16, 1)
    b3c = b3.reshape(1, 1)

    tc_caus = pl.pallas_call(
        _tc_caus_body,
        grid=(TC_B,),
        in_specs=[pl.BlockSpec((1, S, 16, 128), lambda b: (b, 0, 0, 0))],
        out_specs=pl.BlockSpec((1, S, 1), lambda b: (b, 0, 0)),
        out_shape=jax.ShapeDtypeStruct((TC_B, S, 1), F32),
    )(rel_v)

    full = lambda shape: pl.BlockSpec(shape, lambda: (0,) * len(shape))
    cav, cov = pl.pallas_call(
        _main_body,
        in_specs=[
            full((B, 16, 128)), full((B, 16, 128)),
            full((B, S)), full((B, S)),
            full((TC_B, S, 1)), full((NSC, S)),
            full((32, 8)), full((32, 1)),
            full((16, 32)), full((16, 1)),
            full((1, 16)), full((1, 1)),
        ],
        out_specs=(full((B, 16, 128)), full((B, 16, 128))),
        out_shape=(jax.ShapeDtypeStruct((B, 16, 128), F32),
                   jax.ShapeDtypeStruct((B, 16, 128), F32)),
    )(av, ov, ea, eo, tc_caus, sc_caus, w1t, b1c, w2t, b2c, w3t, b3c)

    return _from_view(cav), _from_view(cov)


# speedup vs baseline: 1.3658x; 1.3412x over previous
import functools

import jax
import jax.numpy as jnp
from jax import lax
from jax.experimental import pallas as pl
from jax.experimental.pallas import tpu as pltpu
from jax.experimental.pallas import tpu_sc as plsc

B, S = 8, 512
F32 = jnp.float32

TC_B = 6
NSC = B - TC_B
NW = 32
SLABS = NSC * S
SLABS_PW = SLABS // NW
GULP = 16
NGULP = SLABS_PW // GULP


_GATHER_DNUMS = lax.GatherDimensionNumbers(
    offset_dims=(), collapsed_slice_dims=(0,), start_index_map=(0,))


def _lane_allmax(v, lane):
    for d in (8, 4, 2, 1):
        idx = lane ^ d
        p = lax.gather(v, idx[:, None], _GATHER_DNUMS, slice_sizes=(1,),
                       mode=lax.GatherScatterMode.PROMISE_IN_BOUNDS)
        v = jnp.maximum(v, p)
    return v


def _sc_caus_body(rel_hbm, out_hbm, buf, res):
    wid = lax.axis_index("s") * 2 + lax.axis_index("c")
    slab0 = TC_B * S + wid * SLABS_PW
    lane = lax.broadcasted_iota(jnp.int32, (16,), 0)
    zero = jnp.zeros((16,), F32)

    def gulp_body(g, carry):
        first = slab0 + g * GULP
        pltpu.sync_copy(rel_hbm.at[pl.ds(first * 16, GULP * 16)], buf)

        def slab_body(s, rvec):
            i = (first + s) & (S - 1)
            acc0, acc1 = zero, zero
            for jblk in range(4):
                for lv in range(8):
                    mask = lane < (i - (jblk * 128 + lv * 16))
                    r0 = s * 16 + jblk * 4
                    v0 = buf[r0 + 0, pl.ds(lv * 16, 16)]
                    v1 = buf[r0 + 1, pl.ds(lv * 16, 16)]
                    v2 = buf[r0 + 2, pl.ds(lv * 16, 16)]
                    v3 = buf[r0 + 3, pl.ds(lv * 16, 16)]
                    acc0 = jnp.maximum(acc0,
                                       jnp.where(mask, jnp.maximum(v0, v1), 0.0))
                    acc1 = jnp.maximum(acc1,
                                       jnp.where(mask, jnp.maximum(v2, v3), 0.0))
            acc = jnp.maximum(acc0, acc1)
            amax = _lane_allmax(acc, lane)
            return jnp.where((lane == s) & (amax > 0.5), 1.0, rvec)

        rvec = lax.fori_loop(0, GULP, slab_body, jnp.zeros((16,), F32))
        res[pl.ds(g * GULP, GULP)] = rvec
        return carry

    lax.fori_loop(0, NGULP, gulp_body, 0)
    pltpu.sync_copy(res, out_hbm.at[pl.ds(wid * SLABS_PW, SLABS_PW)])


def _sc_caus(rel_rows):
    mesh = plsc.VectorSubcoreMesh(core_axis_name="c", subcore_axis_name="s")
    k = functools.partial(
        pl.kernel,
        mesh=mesh,
        out_type=jax.ShapeDtypeStruct((SLABS,), F32),
        scratch_types=[
            pltpu.VMEM((GULP * 16, 128), F32),
            pltpu.VMEM((SLABS_PW,), F32),
        ],
    )(_sc_caus_body)
    return k(rel_rows)


def _shift_fwd(x, d, fill, lane):
    r = pltpu.roll(x, d, 1)
    return jnp.where(lane >= d, r, fill)


def _shift_bwd(x, d, fill, lane):
    r = pltpu.roll(x, S - d, 1)
    return jnp.where(lane < S - d, r, fill)


def _channels(view):
    return [jnp.concatenate([view[:, 4 * k + c, :] for k in range(4)], axis=1)
            for c in range(4)]


def _tc_caus_body(rel_ref, out_ref):
    x = rel_ref[0]
    i_iota = jax.lax.broadcasted_iota(jnp.int32, (S, 16, 128), 0)
    m_iota = jax.lax.broadcasted_iota(jnp.int32, (S, 16, 128), 1)
    l_iota = jax.lax.broadcasted_iota(jnp.int32, (S, 16, 128), 2)
    j = ((m_iota >> 2) << 7) + l_iota
    hit = (x > 0.5) & (j < i_iota)
    out_ref[0] = jnp.any(hit, axis=(1, 2)).astype(F32)[:, None]


def _main_body(av_ref, ov_ref, ea_ref, eo_ref, tcc_ref, scc_ref,
               w1t_ref, b1c_ref, w2t_ref, b2c_ref, w3t_ref, b3c_ref,
               cav_ref, cov_ref):
    if True:
        a0, a1, a2, a3 = _channels(av_ref[...])
        o0, o1, o2, o3 = _channels(ov_ref[...])

        feats = [a0, a1, a2, a3, o0, o1, o2, o3]
        xall = jnp.concatenate(
            [jnp.reshape(f, (1, B * S)) for f in feats], axis=0)
        h = jnp.dot(w1t_ref[...], xall, preferred_element_type=F32) + b1c_ref[...]
        h = jnp.maximum(h, 0.0)
        h = jnp.dot(w2t_ref[...], h, preferred_element_type=F32) + b2c_ref[...]
        h = jnp.maximum(h, 0.0)
        z = jnp.dot(w3t_ref[...], h, preferred_element_type=F32) + b3c_ref[...]
        score = jax.nn.sigmoid(jnp.reshape(z, (B, S)))
        f1 = jnp.where(score < 0.5, 2.0 * score, 1.0)

        def imp(c0, c1, c2, c3):
            m = jnp.maximum(jnp.maximum(c0, c1), jnp.maximum(c2, c3))
            e0, e1 = jnp.exp(c0 - m), jnp.exp(c1 - m)
            e2, e3 = jnp.exp(c2 - m), jnp.exp(c3 - m)
            den = ((e0 + e1) + e2) + e3
            return (e0 / den + e1 / den) > 0.5

        imp_asp = imp(a0, a1, a2, a3)
        imp_op = imp(o0, o1, o2, o3)

        ri = jax.lax.broadcasted_iota(jnp.int32, (S, S), 0)
        ci = jax.lax.broadcasted_iota(jnp.int32, (S, S), 1)
        band = (jnp.abs(ri - ci) <= 3).astype(F32)
        ea = (ea_ref[...] > 0).astype(F32)
        eo = (eo_ref[...] > 0).astype(F32)
        near_op = jnp.dot(eo, band, preferred_element_type=F32) > 0.0
        near_as = jnp.dot(ea, band, preferred_element_type=F32) > 0.0

        r2 = imp_asp & (~near_op)
        r3 = imp_op & (~near_as)
        caus = jnp.concatenate(
            [jnp.reshape(tcc_ref[...], (TC_B, S)), scc_ref[...]], axis=0) > 0.5

        w2a = jnp.where(r2, 0.3, 1.0)
        w2o = jnp.where(r3, 0.3, 1.0)
        w7 = jnp.where(caus, 0.7, 1.0)
        u_a0, u_a1 = (a0 * f1) * w2a, (a1 * f1) * w2a
        u_o0, u_o1 = (o0 * f1) * w2o, (o1 * f1) * w2o
        t_a2, t_a3 = a2 * f1, a3 * f1
        t_o2, t_o3 = o2 * f1, o3 * f1

        P = jnp.maximum(jnp.maximum(t_a2, t_a3), jnp.maximum(t_o2, t_o3)) > 0.5
        Q1 = jnp.maximum(jnp.maximum(u_a0 * w7, u_a1 * w7),
                         jnp.maximum(u_o0, u_o1)) > 0.5
        Q0 = jnp.maximum(jnp.maximum((u_a0 * 0.1) * w7, (u_a1 * 0.1) * w7),
                         jnp.maximum(u_o0 * 0.1, u_o1 * 0.1)) > 0.5
        F = jnp.maximum(
            jnp.maximum(jnp.maximum(a0, a1), jnp.maximum(a2, a3)),
            jnp.maximum(jnp.maximum(o0, o1), jnp.maximum(o2, o3))) > 0.5

        Rf = (P | Q0).astype(F32)
        Q1f = Q1.astype(F32)
        Ff = F.astype(F32)

        lane = jax.lax.broadcasted_iota(jnp.int32, (B, S), 1)
        base = jnp.maximum(
            jnp.maximum(_shift_bwd(Ff, 1, 0.0, lane),
                        _shift_bwd(Ff, 2, 0.0, lane)),
            jnp.maximum(_shift_fwd(Rf, 1, 0.0, lane),
                        _shift_fwd(Rf, 2, 0.0, lane)))
        q1s1 = _shift_fwd(Q1f, 1, 0.0, lane)
        q1s2 = _shift_fwd(Q1f, 2, 0.0, lane)

        ones = jnp.ones((B, S), F32)
        zeros = jnp.zeros((B, S), F32)
        a11, a12, a21, a22 = q1s1, q1s2, ones, zeros
        c1, c2 = base, zeros
        d = 1
        while d < S:
            b11 = _shift_fwd(a11, d, 1.0, lane)
            b12 = _shift_fwd(a12, d, 0.0, lane)
            b21 = _shift_fwd(a21, d, 0.0, lane)
            b22 = _shift_fwd(a22, d, 1.0, lane)
            bc1 = _shift_fwd(c1, d, 0.0, lane)
            bc2 = _shift_fwd(c2, d, 0.0, lane)
            n11 = jnp.maximum(a11 * b11, a12 * b21)
            n12 = jnp.maximum(a11 * b12, a12 * b22)
            n21 = jnp.maximum(a21 * b11, a22 * b21)
            n22 = jnp.maximum(a21 * b12, a22 * b22)
            nc1 = jnp.maximum(jnp.maximum(a11 * bc1, a12 * bc2), c1)
            nc2 = jnp.maximum(jnp.maximum(a21 * bc1, a22 * bc2), c2)
            a11, a12, a21, a22, c1, c2 = n11, n12, n21, n22, nc1, nc2
            d *= 2

        iso = jnp.where(c1 > 0.5, 1.0, 0.1)

        outs_a = ((u_a0 * iso) * w7, (u_a1 * iso) * w7, t_a2, t_a3)
        outs_o = (u_o0 * iso, u_o1 * iso, t_o2, t_o3)
        for k in range(4):
            sl = slice(128 * k, 128 * (k + 1))
            for c in range(4):
                cav_ref[:, 4 * k + c, :] = outs_a[c][:, sl]
                cov_ref[:, 4 * k + c, :] = outs_o[c][:, sl]


def _to_view(x):
    return (x.reshape(B, 4, 128, 4)
            .transpose(0, 1, 3, 2)
            .reshape(B, 16, 128))


def _from_view(v):
    return (v.reshape(B, 4, 4, 128)
            .transpose(0, 1, 3, 2)
            .reshape(B, S, 4))


def kernel(aspect_logits, opinion_logits, aspect_opinion_relations,
           explicit_aspects, explicit_opinions, W1, b1, W2, b2, W3, b3):
    rel_v = (aspect_opinion_relations.reshape(B, S, 4, 128, 4)
             .transpose(0, 1, 2, 4, 3)
             .reshape(B, S, 16, 128))
    rel_rows = rel_v.reshape(B * S * 16, 128)
    sc_caus = _sc_caus(rel_rows).reshape(NSC, S)

    av = _to_view(aspect_logits)
    ov = _to_view(opinion_logits)
    ea = explicit_aspects.astype(jnp.int32)
    eo = explicit_opinions.astype(jnp.int32)
    w1t = W1.T
    w2t = W2.T
    w3t = W3.T
    b1c = b1.reshape(32, 1)
    b2c = b2.reshape(16, 1)
    b3c = b3.reshape(1, 1)

    tc_caus = pl.pallas_call(
        _tc_caus_body,
        grid=(TC_B,),
        in_specs=[pl.BlockSpec((1, S, 16, 128), lambda b: (b, 0, 0, 0))],
        out_specs=pl.BlockSpec((1, S, 1), lambda b: (b, 0, 0)),
        out_shape=jax.ShapeDtypeStruct((TC_B, S, 1), F32),
    )(rel_v)

    full = lambda shape: pl.BlockSpec(shape, lambda: (0,) * len(shape))
    cav, cov = pl.pallas_call(
        _main_body,
        in_specs=[
            full((B, 16, 128)), full((B, 16, 128)),
            full((B, S)), full((B, S)),
            full((TC_B, S, 1)), full((NSC, S)),
            full((32, 8)), full((32, 1)),
            full((16, 32)), full((16, 1)),
            full((1, 16)), full((1, 1)),
        ],
        out_specs=(full((B, 16, 128)), full((B, 16, 128))),
        out_shape=(jax.ShapeDtypeStruct((B, 16, 128), F32),
                   jax.ShapeDtypeStruct((B, 16, 128), F32)),
    )(av, ov, ea, eo, tc_caus, sc_caus, w1t, b1c, w2t, b2c, w3t, b3c)

    return _from_view(cav), _from_view(cov)


# speedup vs baseline: 1.6189x; 1.1853x over previous
import functools

import jax
import jax.numpy as jnp
from jax import lax
from jax.experimental import pallas as pl
from jax.experimental.pallas import tpu as pltpu
from jax.experimental.pallas import tpu_sc as plsc

B, S = 8, 512
F32 = jnp.float32

TC_B = 7
NSC = B - TC_B
NW = 32
SLABS = NSC * S
SLABS_PW = SLABS // NW
GULP = 16
NGULP = SLABS_PW // GULP


_GATHER_DNUMS = lax.GatherDimensionNumbers(
    offset_dims=(), collapsed_slice_dims=(0,), start_index_map=(0,))


def _lane_allmax(v, lane):
    for d in (8, 4, 2, 1):
        idx = lane ^ d
        p = lax.gather(v, idx[:, None], _GATHER_DNUMS, slice_sizes=(1,),
                       mode=lax.GatherScatterMode.PROMISE_IN_BOUNDS)
        v = jnp.maximum(v, p)
    return v


def _sc_caus_body(rel_hbm, out_hbm, buf, res):
    wid = lax.axis_index("s") * 2 + lax.axis_index("c")
    slab0 = TC_B * S + wid * SLABS_PW
    lane = lax.broadcasted_iota(jnp.int32, (16,), 0)
    zero = jnp.zeros((16,), F32)

    def gulp_body(g, carry):
        first = slab0 + g * GULP
        pltpu.sync_copy(rel_hbm.at[pl.ds(first * 16, GULP * 16)], buf)

        def slab_body(s, rvec):
            i = (first + s) & (S - 1)
            acc0, acc1 = zero, zero
            for jblk in range(4):
                for lv in range(8):
                    mask = lane < (i - (jblk * 128 + lv * 16))
                    r0 = s * 16 + jblk * 4
                    v0 = buf[r0 + 0, pl.ds(lv * 16, 16)]
                    v1 = buf[r0 + 1, pl.ds(lv * 16, 16)]
                    v2 = buf[r0 + 2, pl.ds(lv * 16, 16)]
                    v3 = buf[r0 + 3, pl.ds(lv * 16, 16)]
                    acc0 = jnp.maximum(acc0,
                                       jnp.where(mask, jnp.maximum(v0, v1), 0.0))
                    acc1 = jnp.maximum(acc1,
                                       jnp.where(mask, jnp.maximum(v2, v3), 0.0))
            acc = jnp.maximum(acc0, acc1)
            amax = _lane_allmax(acc, lane)
            return jnp.where((lane == s) & (amax > 0.5), 1.0, rvec)

        rvec = lax.fori_loop(0, GULP, slab_body, jnp.zeros((16,), F32))
        res[pl.ds(g * GULP, GULP)] = rvec
        return carry

    lax.fori_loop(0, NGULP, gulp_body, 0)
    pltpu.sync_copy(res, out_hbm.at[pl.ds(wid * SLABS_PW, SLABS_PW)])


def _sc_caus(rel_rows):
    mesh = plsc.VectorSubcoreMesh(core_axis_name="c", subcore_axis_name="s")
    k = functools.partial(
        pl.kernel,
        mesh=mesh,
        out_type=jax.ShapeDtypeStruct((SLABS,), F32),
        scratch_types=[
            pltpu.VMEM((GULP * 16, 128), F32),
            pltpu.VMEM((SLABS_PW,), F32),
        ],
    )(_sc_caus_body)
    return k(rel_rows)


def _shift_fwd(x, d, fill, lane):
    r = pltpu.roll(x, d, 1)
    return jnp.where(lane >= d, r, fill)


def _shift_bwd(x, d, fill, lane):
    r = pltpu.roll(x, S - d, 1)
    return jnp.where(lane < S - d, r, fill)


def _channels(view):
    return [jnp.concatenate([view[:, 4 * k + c, :] for k in range(4)], axis=1)
            for c in range(4)]


def _tc_caus_body(rel_ref, out_ref):
    x = rel_ref[0]
    i_iota = jax.lax.broadcasted_iota(jnp.int32, (S, 16, 128), 0)
    m_iota = jax.lax.broadcasted_iota(jnp.int32, (S, 16, 128), 1)
    l_iota = jax.lax.broadcasted_iota(jnp.int32, (S, 16, 128), 2)
    j = ((m_iota >> 2) << 7) + l_iota
    hit = (x > 0.5) & (j < i_iota)
    out_ref[0] = jnp.any(hit, axis=(1, 2)).astype(F32)[:, None]


def _main_body(av_ref, ov_ref, ea_ref, eo_ref, tcc_ref, scc_ref,
               w1t_ref, b1c_ref, w2t_ref, b2c_ref, w3t_ref, b3c_ref,
               cav_ref, cov_ref):
    if True:
        a0, a1, a2, a3 = _channels(av_ref[...])
        o0, o1, o2, o3 = _channels(ov_ref[...])

        feats = [a0, a1, a2, a3, o0, o1, o2, o3]
        xall = jnp.concatenate(
            [jnp.reshape(f, (1, B * S)) for f in feats], axis=0)
        h = jnp.dot(w1t_ref[...], xall, preferred_element_type=F32) + b1c_ref[...]
        h = jnp.maximum(h, 0.0)
        h = jnp.dot(w2t_ref[...], h, preferred_element_type=F32) + b2c_ref[...]
        h = jnp.maximum(h, 0.0)
        z = jnp.dot(w3t_ref[...], h, preferred_element_type=F32) + b3c_ref[...]
        score = jax.nn.sigmoid(jnp.reshape(z, (B, S)))
        f1 = jnp.where(score < 0.5, 2.0 * score, 1.0)

        def imp(c0, c1, c2, c3):
            m = jnp.maximum(jnp.maximum(c0, c1), jnp.maximum(c2, c3))
            e0, e1 = jnp.exp(c0 - m), jnp.exp(c1 - m)
            e2, e3 = jnp.exp(c2 - m), jnp.exp(c3 - m)
            den = ((e0 + e1) + e2) + e3
            return (e0 / den + e1 / den) > 0.5

        imp_asp = imp(a0, a1, a2, a3)
        imp_op = imp(o0, o1, o2, o3)

        ri = jax.lax.broadcasted_iota(jnp.int32, (S, S), 0)
        ci = jax.lax.broadcasted_iota(jnp.int32, (S, S), 1)
        band = (jnp.abs(ri - ci) <= 3).astype(F32)
        ea = (ea_ref[...] > 0).astype(F32)
        eo = (eo_ref[...] > 0).astype(F32)
        near_op = jnp.dot(eo, band, preferred_element_type=F32) > 0.0
        near_as = jnp.dot(ea, band, preferred_element_type=F32) > 0.0

        r2 = imp_asp & (~near_op)
        r3 = imp_op & (~near_as)
        caus = jnp.concatenate(
            [jnp.reshape(tcc_ref[...], (TC_B, S)), scc_ref[...]], axis=0) > 0.5

        w2a = jnp.where(r2, 0.3, 1.0)
        w2o = jnp.where(r3, 0.3, 1.0)
        w7 = jnp.where(caus, 0.7, 1.0)
        u_a0, u_a1 = (a0 * f1) * w2a, (a1 * f1) * w2a
        u_o0, u_o1 = (o0 * f1) * w2o, (o1 * f1) * w2o
        t_a2, t_a3 = a2 * f1, a3 * f1
        t_o2, t_o3 = o2 * f1, o3 * f1

        P = jnp.maximum(jnp.maximum(t_a2, t_a3), jnp.maximum(t_o2, t_o3)) > 0.5
        Q1 = jnp.maximum(jnp.maximum(u_a0 * w7, u_a1 * w7),
                         jnp.maximum(u_o0, u_o1)) > 0.5
        Q0 = jnp.maximum(jnp.maximum((u_a0 * 0.1) * w7, (u_a1 * 0.1) * w7),
                         jnp.maximum(u_o0 * 0.1, u_o1 * 0.1)) > 0.5
        F = jnp.maximum(
            jnp.maximum(jnp.maximum(a0, a1), jnp.maximum(a2, a3)),
            jnp.maximum(jnp.maximum(o0, o1), jnp.maximum(o2, o3))) > 0.5

        Rf = (P | Q0).astype(F32)
        Q1f = Q1.astype(F32)
        Ff = F.astype(F32)

        lane = jax.lax.broadcasted_iota(jnp.int32, (B, S), 1)
        base = jnp.maximum(
            jnp.maximum(_shift_bwd(Ff, 1, 0.0, lane),
                        _shift_bwd(Ff, 2, 0.0, lane)),
            jnp.maximum(_shift_fwd(Rf, 1, 0.0, lane),
                        _shift_fwd(Rf, 2, 0.0, lane)))
        q1s1 = _shift_fwd(Q1f, 1, 0.0, lane)
        q1s2 = _shift_fwd(Q1f, 2, 0.0, lane)

        ones = jnp.ones((B, S), F32)
        zeros = jnp.zeros((B, S), F32)
        a11, a12, a21, a22 = q1s1, q1s2, ones, zeros
        c1, c2 = base, zeros
        d = 1
        while d < S:
            b11 = _shift_fwd(a11, d, 1.0, lane)
            b12 = _shift_fwd(a12, d, 0.0, lane)
            b21 = _shift_fwd(a21, d, 0.0, lane)
            b22 = _shift_fwd(a22, d, 1.0, lane)
            bc1 = _shift_fwd(c1, d, 0.0, lane)
            bc2 = _shift_fwd(c2, d, 0.0, lane)
            n11 = jnp.maximum(a11 * b11, a12 * b21)
            n12 = jnp.maximum(a11 * b12, a12 * b22)
            n21 = jnp.maximum(a21 * b11, a22 * b21)
            n22 = jnp.maximum(a21 * b12, a22 * b22)
            nc1 = jnp.maximum(jnp.maximum(a11 * bc1, a12 * bc2), c1)
            nc2 = jnp.maximum(jnp.maximum(a21 * bc1, a22 * bc2), c2)
            a11, a12, a21, a22, c1, c2 = n11, n12, n21, n22, nc1, nc2
            d *= 2

        iso = jnp.where(c1 > 0.5, 1.0, 0.1)

        outs_a = ((u_a0 * iso) * w7, (u_a1 * iso) * w7, t_a2, t_a3)
        outs_o = (u_o0 * iso, u_o1 * iso, t_o2, t_o3)
        for k in range(4):
            sl = slice(128 * k, 128 * (k + 1))
            for c in range(4):
                cav_ref[:, 4 * k + c, :] = outs_a[c][:, sl]
                cov_ref[:, 4 * k + c, :] = outs_o[c][:, sl]


def _to_view(x):
    return (x.reshape(B, 4, 128, 4)
            .transpose(0, 1, 3, 2)
            .reshape(B, 16, 128))


def _from_view(v):
    return (v.reshape(B, 4, 4, 128)
            .transpose(0, 1, 3, 2)
            .reshape(B, S, 4))


def kernel(aspect_logits, opinion_logits, aspect_opinion_relations,
           explicit_aspects, explicit_opinions, W1, b1, W2, b2, W3, b3):
    rel_v = (aspect_opinion_relations.reshape(B, S, 4, 128, 4)
             .transpose(0, 1, 2, 4, 3)
             .reshape(B, S, 16, 128))
    rel_rows = rel_v.reshape(B * S * 16, 128)
    sc_caus = _sc_caus(rel_rows).reshape(NSC, S)

    av = _to_view(aspect_logits)
    ov = _to_view(opinion_logits)
    ea = explicit_aspects.astype(jnp.int32)
    eo = explicit_opinions.astype(jnp.int32)
    w1t = W1.T
    w2t = W2.T
    w3t = W3.T
    b1c = b1.reshape(32, 1)
    b2c = b2.reshape(16, 1)
    b3c = b3.reshape(1, 1)

    tc_caus = pl.pallas_call(
        _tc_caus_body,
        grid=(TC_B,),
        in_specs=[pl.BlockSpec((1, S, 16, 128), lambda b: (b, 0, 0, 0))],
        out_specs=pl.BlockSpec((1, S, 1), lambda b: (b, 0, 0)),
        out_shape=jax.ShapeDtypeStruct((TC_B, S, 1), F32),
    )(rel_v)

    full = lambda shape: pl.BlockSpec(shape, lambda: (0,) * len(shape))
    cav, cov = pl.pallas_call(
        _main_body,
        in_specs=[
            full((B, 16, 128)), full((B, 16, 128)),
            full((B, S)), full((B, S)),
            full((TC_B, S, 1)), full((NSC, S)),
            full((32, 8)), full((32, 1)),
            full((16, 32)), full((16, 1)),
            full((1, 16)), full((1, 1)),
        ],
        out_specs=(full((B, 16, 128)), full((B, 16, 128))),
        out_shape=(jax.ShapeDtypeStruct((B, 16, 128), F32),
                   jax.ShapeDtypeStruct((B, 16, 128), F32)),
    )(av, ov, ea, eo, tc_caus, sc_caus, w1t, b1c, w2t, b2c, w3t, b3c)

    return _from_view(cav), _from_view(cov)


# speedup vs baseline: 2.6703x; 1.6495x over previous
import jax
import jax.numpy as jnp
from jax.experimental import pallas as pl
from jax.experimental.pallas import tpu as pltpu

B, S = 8, 512
F32 = jnp.float32


def _shift_fwd(x, d, fill, lane):
    r = pltpu.roll(x, d, 1)
    return jnp.where(lane >= d, r, fill)


def _shift_bwd(x, d, fill, lane):
    r = pltpu.roll(x, S - d, 1)
    return jnp.where(lane < S - d, r, fill)


def _channels(view):
    return [jnp.concatenate([view[:, 4 * k + c, :] for k in range(4)], axis=1)
            for c in range(4)]


def _body(rel_ref, av_ref, ov_ref, ea_ref, eo_ref,
          w1t_ref, b1c_ref, w2t_ref, b2c_ref, w3t_ref, b3c_ref,
          cav_ref, cov_ref, caus_sc):
    b = pl.program_id(0)

    x = rel_ref[0]
    i_iota = jax.lax.broadcasted_iota(jnp.int32, (S, 16, 128), 0)
    m_iota = jax.lax.broadcasted_iota(jnp.int32, (S, 16, 128), 1)
    l_iota = jax.lax.broadcasted_iota(jnp.int32, (S, 16, 128), 2)
    j = ((m_iota >> 2) << 7) + l_iota
    hit = (x > 0.5) & (j < i_iota)
    caus_sc[b] = jnp.any(hit, axis=(1, 2)).astype(F32)[:, None]

    @pl.when(b == B - 1)
    def _main():
        a0, a1, a2, a3 = _channels(av_ref[...])
        o0, o1, o2, o3 = _channels(ov_ref[...])

        feats = [a0, a1, a2, a3, o0, o1, o2, o3]
        xall = jnp.concatenate(
            [jnp.reshape(f, (1, B * S)) for f in feats], axis=0)
        h = jnp.dot(w1t_ref[...], xall, preferred_element_type=F32) + b1c_ref[...]
        h = jnp.maximum(h, 0.0)
        h = jnp.dot(w2t_ref[...], h, preferred_element_type=F32) + b2c_ref[...]
        h = jnp.maximum(h, 0.0)
        z = jnp.dot(w3t_ref[...], h, preferred_element_type=F32) + b3c_ref[...]
        score = jax.nn.sigmoid(jnp.reshape(z, (B, S)))
        f1 = jnp.where(score < 0.5, 2.0 * score, 1.0)

        def imp(c0, c1, c2, c3):
            m = jnp.maximum(jnp.maximum(c0, c1), jnp.maximum(c2, c3))
            e0, e1 = jnp.exp(c0 - m), jnp.exp(c1 - m)
            e2, e3 = jnp.exp(c2 - m), jnp.exp(c3 - m)
            den = ((e0 + e1) + e2) + e3
            return (e0 / den + e1 / den) > 0.5

        imp_asp = imp(a0, a1, a2, a3)
        imp_op = imp(o0, o1, o2, o3)

        ri = jax.lax.broadcasted_iota(jnp.int32, (S, S), 0)
        ci = jax.lax.broadcasted_iota(jnp.int32, (S, S), 1)
        band = (jnp.abs(ri - ci) <= 3).astype(F32)
        ea = (ea_ref[...] > 0).astype(F32)
        eo = (eo_ref[...] > 0).astype(F32)
        near_op = jnp.dot(eo, band, preferred_element_type=F32) > 0.0
        near_as = jnp.dot(ea, band, preferred_element_type=F32) > 0.0

        r2 = imp_asp & (~near_op)
        r3 = imp_op & (~near_as)
        caus = jnp.reshape(caus_sc[...], (B, S)) > 0.5

        w2a = jnp.where(r2, 0.3, 1.0)
        w2o = jnp.where(r3, 0.3, 1.0)
        w7 = jnp.where(caus, 0.7, 1.0)
        u_a0, u_a1 = (a0 * f1) * w2a, (a1 * f1) * w2a
        u_o0, u_o1 = (o0 * f1) * w2o, (o1 * f1) * w2o
        t_a2, t_a3 = a2 * f1, a3 * f1
        t_o2, t_o3 = o2 * f1, o3 * f1

        P = jnp.maximum(jnp.maximum(t_a2, t_a3), jnp.maximum(t_o2, t_o3)) > 0.5
        Q1 = jnp.maximum(jnp.maximum(u_a0 * w7, u_a1 * w7),
                         jnp.maximum(u_o0, u_o1)) > 0.5
        Q0 = jnp.maximum(jnp.maximum((u_a0 * 0.1) * w7, (u_a1 * 0.1) * w7),
                         jnp.maximum(u_o0 * 0.1, u_o1 * 0.1)) > 0.5
        F = jnp.maximum(
            jnp.maximum(jnp.maximum(a0, a1), jnp.maximum(a2, a3)),
            jnp.maximum(jnp.maximum(o0, o1), jnp.maximum(o2, o3))) > 0.5

        Rf = (P | Q0).astype(F32)
        Q1f = Q1.astype(F32)
        Ff = F.astype(F32)

        lane = jax.lax.broadcasted_iota(jnp.int32, (B, S), 1)
        base = jnp.maximum(
            jnp.maximum(_shift_bwd(Ff, 1, 0.0, lane),
                        _shift_bwd(Ff, 2, 0.0, lane)),
            jnp.maximum(_shift_fwd(Rf, 1, 0.0, lane),
                        _shift_fwd(Rf, 2, 0.0, lane)))
        q1s1 = _shift_fwd(Q1f, 1, 0.0, lane)
        q1s2 = _shift_fwd(Q1f, 2, 0.0, lane)

        ones = jnp.ones((B, S), F32)
        zeros = jnp.zeros((B, S), F32)
        a11, a12, a21, a22 = q1s1, q1s2, ones, zeros
        c1, c2 = base, zeros
        d = 1
        while d < S:
            b11 = _shift_fwd(a11, d, 1.0, lane)
            b12 = _shift_fwd(a12, d, 0.0, lane)
            b21 = _shift_fwd(a21, d, 0.0, lane)
            b22 = _shift_fwd(a22, d, 1.0, lane)
            bc1 = _shift_fwd(c1, d, 0.0, lane)
            bc2 = _shift_fwd(c2, d, 0.0, lane)
            n11 = jnp.maximum(a11 * b11, a12 * b21)
            n12 = jnp.maximum(a11 * b12, a12 * b22)
            n21 = jnp.maximum(a21 * b11, a22 * b21)
            n22 = jnp.maximum(a21 * b12, a22 * b22)
            nc1 = jnp.maximum(jnp.maximum(a11 * bc1, a12 * bc2), c1)
            nc2 = jnp.maximum(jnp.maximum(a21 * bc1, a22 * bc2), c2)
            a11, a12, a21, a22, c1, c2 = n11, n12, n21, n22, nc1, nc2
            d *= 2

        iso = jnp.where(c1 > 0.5, 1.0, 0.1)

        outs_a = ((u_a0 * iso) * w7, (u_a1 * iso) * w7, t_a2, t_a3)
        outs_o = (u_o0 * iso, u_o1 * iso, t_o2, t_o3)
        for k in range(4):
            sl = slice(128 * k, 128 * (k + 1))
            for c in range(4):
                cav_ref[:, 4 * k + c, :] = outs_a[c][:, sl]
                cov_ref[:, 4 * k + c, :] = outs_o[c][:, sl]


def _to_view(x):
    return (x.reshape(B, 4, 128, 4)
            .transpose(0, 1, 3, 2)
            .reshape(B, 16, 128))


def _from_view(v):
    return (v.reshape(B, 4, 4, 128)
            .transpose(0, 1, 3, 2)
            .reshape(B, S, 4))


def kernel(aspect_logits, opinion_logits, aspect_opinion_relations,
           explicit_aspects, explicit_opinions, W1, b1, W2, b2, W3, b3):
    rel_v = (aspect_opinion_relations.reshape(B, S, 4, 128, 4)
             .transpose(0, 1, 2, 4, 3)
             .reshape(B, S, 16, 128))
    av = _to_view(aspect_logits)
    ov = _to_view(opinion_logits)
    ea = explicit_aspects.astype(jnp.int32)
    eo = explicit_opinions.astype(jnp.int32)
    w1t = W1.T
    w2t = W2.T
    w3t = W3.T
    b1c = b1.reshape(32, 1)
    b2c = b2.reshape(16, 1)
    b3c = b3.reshape(1, 1)

    full = lambda shape: pl.BlockSpec(shape, lambda b: (0,) * len(shape))
    cav, cov = pl.pallas_call(
        _body,
        grid=(B,),
        in_specs=[
            pl.BlockSpec((1, S, 16, 128), lambda b: (b, 0, 0, 0)),
            full((B, 16, 128)), full((B, 16, 128)),
            full((B, S)), full((B, S)),
            full((32, 8)), full((32, 1)),
            full((16, 32)), full((16, 1)),
            full((1, 16)), full((1, 1)),
        ],
        out_specs=(full((B, 16, 128)), full((B, 16, 128))),
        out_shape=(jax.ShapeDtypeStruct((B, 16, 128), F32),
                   jax.ShapeDtypeStruct((B, 16, 128), F32)),
        scratch_shapes=[pltpu.VMEM((B, S, 1), F32)],
    )(rel_v, av, ov, ea, eo, w1t, b1c, w2t, b2c, w3t, b3c)

    return _from_view(cav), _from_view(cov)
